# baseline (speedup 1.0000x reference)
import functools

import jax
import jax.numpy as jnp
from jax import lax
from jax.experimental import pallas as pl
from jax.experimental.pallas import tpu as pltpu
from jax.experimental.pallas import tpu_sc as plsc

SEQ_LEN = 8192
NUM_VALUES = 21
NUM_CORES = 1
NUM_SUBCORES = 16
LANES = 16
NUM_WORKERS = NUM_CORES * NUM_SUBCORES
ROWS_PER_TILE = SEQ_LEN // NUM_WORKERS
CHUNKS = ROWS_PER_TILE // LANES
QUARTERS = 4

_LN2 = 0.6931471805599453
_SQRT2 = 1.4142135623730951


def _vlog(x):
    bits = plsc.bitcast(x, jnp.int32)
    e = lax.shift_right_logical(bits, 23) - 127
    mbits = (bits & 0x007FFFFF) | 0x3F800000
    m = plsc.bitcast(mbits, jnp.float32)
    big = m >= _SQRT2
    m = jnp.where(big, m * 0.5, m)
    e = jnp.where(big, e + 1, e)
    s = (m - 1.0) / (m + 1.0)
    s2 = s * s
    ln_m = 2.0 * s * (1.0 + s2 * (1.0 / 3.0 + s2 * (0.2 + s2 * (1.0 / 7.0))))
    return e.astype(jnp.float32) * _LN2 + ln_m


_mesh = plsc.VectorSubcoreMesh(
    core_axis_name="c", subcore_axis_name="s",
    num_cores=NUM_CORES, num_subcores=NUM_SUBCORES,
)


@functools.partial(
    pl.kernel,
    out_type=jax.ShapeDtypeStruct((LANES,), jnp.float32),
    mesh=_mesh,
    compiler_params=pltpu.CompilerParams(needs_layout_passes=False),
    scratch_types=[
        pltpu.VMEM((ROWS_PER_TILE,), jnp.int32),
        pltpu.VMEM((ROWS_PER_TILE * NUM_VALUES,), jnp.float32),
        pltpu.VMEM((LANES,), jnp.float32),
        pltpu.VMEM_SHARED((NUM_SUBCORES * LANES,), jnp.float32),
        pltpu.VMEM((NUM_SUBCORES * LANES,), jnp.float32),
        pltpu.VMEM((LANES,), jnp.float32),
        pltpu.SemaphoreType.DMA,
        pltpu.SemaphoreType.DMA,
        pltpu.SemaphoreType.DMA,
        pltpu.SemaphoreType.DMA,
        pltpu.SemaphoreType.DMA,
    ],
)
def _sc_logprob(seq_hbm, tab_hbm, out_hbm, seq_v, rows_v, acc_v,
                shared_sp, gath_v, res_v, sem_s, sem_q0, sem_q1, sem_q2,
                sem_q3):
    sid = lax.axis_index("s")
    base = sid * ROWS_PER_TILE
    qrows = ROWS_PER_TILE // QUARTERS
    qwords = qrows * NUM_VALUES
    cp_seq = pltpu.async_copy(
        seq_hbm.at[pl.ds(base, ROWS_PER_TILE)], seq_v, sem_s)
    cps = []
    for q, sem in enumerate((sem_q0, sem_q1, sem_q2, sem_q3)):
        cps.append(pltpu.async_copy(
            tab_hbm.at[pl.ds(base * NUM_VALUES + q * qwords, qwords)],
            rows_v.at[pl.ds(q * qwords, qwords)], sem))
    cp_seq.wait()
    acc = jnp.zeros((LANES,), jnp.float32)
    for q in range(QUARTERS):
        cps[q].wait()

        def _chunk(jq, a, q=q):
            j = q * (CHUNKS // QUARTERS) + jq
            cols = seq_v[pl.ds(j * LANES, LANES)]
            rows = lax.iota(jnp.int32, LANES) + (j * LANES)
            vals = plsc.load_gather(rows_v, [rows * NUM_VALUES + cols])
            return a + _vlog(vals)

        acc = lax.fori_loop(0, CHUNKS // QUARTERS, _chunk, acc)
    acc_v[...] = acc
    pltpu.sync_copy(acc_v, shared_sp.at[pl.ds(sid * LANES, LANES)])
    plsc.subcore_barrier()

    @pl.when(sid == 0)
    def _():
        pltpu.sync_copy(shared_sp, gath_v)
        total = jnp.zeros((LANES,), jnp.float32)
        for k in range(NUM_SUBCORES):
            total = total + gath_v[pl.ds(k * LANES, LANES)]
        res_v[...] = jnp.broadcast_to(-jnp.sum(total), (LANES,))
        pltpu.sync_copy(res_v, out_hbm)


def kernel(sequence, site_probabilities):
    res = _sc_logprob(
        sequence.astype(jnp.int32), site_probabilities.reshape(-1))
    return res[0]


# speedup vs baseline: 1.0143x; 1.0143x over previous
import functools

import jax
import jax.numpy as jnp
from jax import lax
from jax.experimental import pallas as pl
from jax.experimental.pallas import tpu as pltpu
from jax.experimental.pallas import tpu_sc as plsc

SEQ_LEN = 8192
NUM_VALUES = 21
NUM_CORES = 1
NUM_SUBCORES = 16
LANES = 16
NUM_WORKERS = NUM_CORES * NUM_SUBCORES
ROWS_PER_TILE = SEQ_LEN // NUM_WORKERS
CHUNKS = ROWS_PER_TILE // LANES
QUARTERS = 4

_LN2 = 0.6931471805599453
_LN_COEFFS = (
    -0.017029610590466433, 0.1837008411638296, -0.8520795951885867,
    2.2269434608355745, -3.6471203953770273, 4.205234841506999,
    -2.0996478486876624,
)


def _vlog(x):
    bits = plsc.bitcast(x, jnp.int32)
    e = lax.shift_right_logical(bits, 23) - 127
    mbits = (bits & 0x007FFFFF) | 0x3F800000
    m = plsc.bitcast(mbits, jnp.float32)
    p = jnp.float32(_LN_COEFFS[0])
    for c in _LN_COEFFS[1:]:
        p = p * m + jnp.float32(c)
    return e.astype(jnp.float32) * _LN2 + p


_mesh = plsc.VectorSubcoreMesh(
    core_axis_name="c", subcore_axis_name="s",
    num_cores=NUM_CORES, num_subcores=NUM_SUBCORES,
)


@functools.partial(
    pl.kernel,
    out_type=jax.ShapeDtypeStruct((LANES,), jnp.float32),
    mesh=_mesh,
    compiler_params=pltpu.CompilerParams(needs_layout_passes=False),
    scratch_types=[
        pltpu.VMEM((ROWS_PER_TILE,), jnp.int32),
        pltpu.VMEM((ROWS_PER_TILE * NUM_VALUES,), jnp.float32),
        pltpu.VMEM((LANES,), jnp.float32),
        pltpu.VMEM_SHARED((NUM_SUBCORES * LANES,), jnp.float32),
        pltpu.VMEM((NUM_SUBCORES * LANES,), jnp.float32),
        pltpu.VMEM((LANES,), jnp.float32),
        pltpu.SemaphoreType.DMA,
        pltpu.SemaphoreType.DMA,
        pltpu.SemaphoreType.DMA,
        pltpu.SemaphoreType.DMA,
        pltpu.SemaphoreType.DMA,
    ],
)
def _sc_logprob(seq_hbm, tab_hbm, out_hbm, seq_v, rows_v, acc_v,
                shared_sp, gath_v, res_v, sem_s, sem_q0, sem_q1, sem_q2,
                sem_q3):
    sid = lax.axis_index("s")
    base = sid * ROWS_PER_TILE
    qrows = ROWS_PER_TILE // QUARTERS
    qwords = qrows * NUM_VALUES
    cp_seq = pltpu.async_copy(
        seq_hbm.at[pl.ds(base, ROWS_PER_TILE)], seq_v, sem_s)
    cps = []
    for q, sem in enumerate((sem_q0, sem_q1, sem_q2, sem_q3)):
        cps.append(pltpu.async_copy(
            tab_hbm.at[pl.ds(base * NUM_VALUES + q * qwords, qwords)],
            rows_v.at[pl.ds(q * qwords, qwords)], sem))
    cp_seq.wait()
    acc = jnp.zeros((LANES,), jnp.float32)
    for q in range(QUARTERS):
        cps[q].wait()
        for jq in range(CHUNKS // QUARTERS):
            j = q * (CHUNKS // QUARTERS) + jq
            cols = seq_v[pl.ds(j * LANES, LANES)]
            rows = lax.iota(jnp.int32, LANES) + (j * LANES)
            vals = plsc.load_gather(rows_v, [rows * NUM_VALUES + cols])
            acc = acc + _vlog(vals)
    acc_v[...] = acc
    pltpu.sync_copy(acc_v, shared_sp.at[pl.ds(sid * LANES, LANES)])
    plsc.subcore_barrier()

    @pl.when(sid == 0)
    def _():
        pltpu.sync_copy(shared_sp, gath_v)
        total = jnp.zeros((LANES,), jnp.float32)
        for k in range(NUM_SUBCORES):
            total = total + gath_v[pl.ds(k * LANES, LANES)]
        res_v[...] = jnp.broadcast_to(-jnp.sum(total), (LANES,))
        pltpu.sync_copy(res_v, out_hbm)


def kernel(sequence, site_probabilities):
    res = _sc_logprob(
        sequence.astype(jnp.int32), site_probabilities.reshape(-1))
    return res[0]


# speedup vs baseline: 1.0162x; 1.0019x over previous
import functools

import jax
import jax.numpy as jnp
from jax import lax
from jax.experimental import pallas as pl
from jax.experimental.pallas import tpu as pltpu
from jax.experimental.pallas import tpu_sc as plsc

SEQ_LEN = 8192
NUM_VALUES = 21
NUM_CORES = 1
NUM_SUBCORES = 16
LANES = 16
NUM_WORKERS = NUM_CORES * NUM_SUBCORES
ROWS_PER_TILE = SEQ_LEN // NUM_WORKERS
CHUNKS = ROWS_PER_TILE // LANES
QUARTERS = 2

_LN2 = 0.6931471805599453
_LN_COEFFS = (
    -0.017029610590466433, 0.1837008411638296, -0.8520795951885867,
    2.2269434608355745, -3.6471203953770273, 4.205234841506999,
    -2.0996478486876624,
)


def _vlog(x):
    bits = plsc.bitcast(x, jnp.int32)
    e = lax.shift_right_logical(bits, 23) - 127
    mbits = (bits & 0x007FFFFF) | 0x3F800000
    m = plsc.bitcast(mbits, jnp.float32)
    p = jnp.float32(_LN_COEFFS[0])
    for c in _LN_COEFFS[1:]:
        p = p * m + jnp.float32(c)
    return e.astype(jnp.float32) * _LN2 + p


_mesh = plsc.VectorSubcoreMesh(
    core_axis_name="c", subcore_axis_name="s",
    num_cores=NUM_CORES, num_subcores=NUM_SUBCORES,
)


@functools.partial(
    pl.kernel,
    out_type=jax.ShapeDtypeStruct((LANES,), jnp.float32),
    mesh=_mesh,
    compiler_params=pltpu.CompilerParams(needs_layout_passes=False),
    scratch_types=[
        pltpu.VMEM((ROWS_PER_TILE,), jnp.int32),
        pltpu.VMEM((ROWS_PER_TILE * NUM_VALUES,), jnp.float32),
        pltpu.VMEM((LANES,), jnp.float32),
        pltpu.VMEM_SHARED((NUM_SUBCORES * LANES,), jnp.float32),
        pltpu.VMEM((NUM_SUBCORES * LANES,), jnp.float32),
        pltpu.VMEM((LANES,), jnp.float32),
        pltpu.SemaphoreType.DMA,
        pltpu.SemaphoreType.DMA,
        pltpu.SemaphoreType.DMA,
        pltpu.SemaphoreType.DMA,
        pltpu.SemaphoreType.DMA,
    ],
)
def _sc_logprob(seq_hbm, tab_hbm, out_hbm, seq_v, rows_v, acc_v,
                shared_sp, gath_v, res_v, sem_s, sem_q0, sem_q1, sem_q2,
                sem_q3):
    sid = lax.axis_index("s")
    base = sid * ROWS_PER_TILE
    qrows = ROWS_PER_TILE // QUARTERS
    qwords = qrows * NUM_VALUES
    cp_seq = pltpu.async_copy(
        seq_hbm.at[pl.ds(base, ROWS_PER_TILE)], seq_v, sem_s)
    cps = []
    for q, sem in enumerate((sem_q0, sem_q1, sem_q2, sem_q3)[:QUARTERS]):
        cps.append(pltpu.async_copy(
            tab_hbm.at[pl.ds(base * NUM_VALUES + q * qwords, qwords)],
            rows_v.at[pl.ds(q * qwords, qwords)], sem))
    cp_seq.wait()
    acc = jnp.zeros((LANES,), jnp.float32)
    for q in range(QUARTERS):
        cps[q].wait()
        for jq in range(CHUNKS // QUARTERS):
            j = q * (CHUNKS // QUARTERS) + jq
            cols = seq_v[pl.ds(j * LANES, LANES)]
            rows = lax.iota(jnp.int32, LANES) + (j * LANES)
            vals = plsc.load_gather(rows_v, [rows * NUM_VALUES + cols])
            acc = acc + _vlog(vals)
    acc_v[...] = acc
    pltpu.sync_copy(acc_v, shared_sp.at[pl.ds(sid * LANES, LANES)])
    plsc.subcore_barrier()

    @pl.when(sid == 0)
    def _():
        pltpu.sync_copy(shared_sp, gath_v)
        total = jnp.zeros((LANES,), jnp.float32)
        for k in range(NUM_SUBCORES):
            total = total + gath_v[pl.ds(k * LANES, LANES)]
        res_v[...] = jnp.broadcast_to(-jnp.sum(total), (LANES,))
        pltpu.sync_copy(res_v, out_hbm)


def kernel(sequence, site_probabilities):
    res = _sc_logprob(
        sequence.astype(jnp.int32), site_probabilities.reshape(-1))
    return res[0]
